# baseline (speedup 1.0000x reference)
import functools

import jax
import jax.numpy as jnp
from jax import lax
from jax.experimental import pallas as pl
from jax.experimental.pallas import tpu as pltpu
from jax.experimental.pallas import tpu_sc as plsc

N = 10000
E = 320000
D_IN = 128
DH = 16
NC = 2
NS = 16
NW = NC * NS
CHUNK = 128
ER = 2 * E // CHUNK
CB = 78
DROW = E // CHUNK
N_ACC = 10112
RPT = N_ACC // NS
PK = 8
NP = N // PK
NAP = N_ACC // PK

_mesh = plsc.VectorSubcoreMesh(core_axis_name="c", subcore_axis_name="s")
_acc_ty = jax.ShapeDtypeStruct((NC, N_ACC, DH), jnp.float32)
_sc_params = pltpu.CompilerParams(use_tc_tiling_on_sc=False)


@functools.partial(
    pl.kernel,
    out_type=_acc_ty,
    mesh=_mesh,
    scratch_types=[
        pltpu.VMEM((CB + 1, CHUNK), jnp.int32),
        pltpu.VMEM((CHUNK, DH), jnp.float32),
        pltpu.VMEM_SHARED((N_ACC, DH), jnp.float32),
        pltpu.SemaphoreType.DMA,
        pltpu.SemaphoreType.DMA,
    ],
    compiler_params=_sc_params,
)
def _sc_degree(e_hbm, ones_hbm, zeros_hbm, out_hbm, dst_v, ones_v, acc,
               ss0, ss1):
    cid = lax.axis_index("c")
    sid = lax.axis_index("s")
    wid = sid * NC + cid
    r0 = sid * RPT
    pltpu.sync_copy(zeros_hbm.at[pl.ds(r0, RPT)], acc.at[pl.ds(r0, RPT)])
    pltpu.sync_copy(e_hbm.at[pl.ds(DROW + wid * CB, CB)],
                    dst_v.at[pl.ds(0, CB)])

    @pl.when(wid < 4)
    def _():
        pltpu.sync_copy(e_hbm.at[pl.ds(DROW + NW * CB + wid, 1)],
                        dst_v.at[pl.ds(CB, 1)])

    pltpu.sync_copy(ones_hbm, ones_v)
    plsc.subcore_barrier()

    sems = (ss0, ss1)
    for b in range(2):
        pltpu.async_copy(ones_v, acc.at[dst_v.at[b]], sems[b], add=True)

    def body(i, carry):
        j0 = 2 * i
        for b in range(2):
            j = j0 + b
            pltpu.make_async_copy(
                ones_v, acc.at[dst_v.at[j - 2]], sems[b]).wait()
            pltpu.async_copy(ones_v, acc.at[dst_v.at[j]], sems[b], add=True)
        return carry

    lax.fori_loop(1, CB // 2, body, 0)
    for b in range(2):
        pltpu.make_async_copy(
            ones_v, acc.at[dst_v.at[CB - 2 + b]], sems[b]).wait()

    @pl.when(wid < 4)
    def _():
        pltpu.sync_copy(ones_v, acc.at[dst_v.at[CB]], add=True)

    plsc.subcore_barrier()
    pltpu.sync_copy(acc.at[pl.ds(r0, RPT)], out_hbm.at[cid, pl.ds(r0, RPT)])


@functools.partial(
    pl.kernel,
    out_type=_acc_ty,
    mesh=_mesh,
    scratch_types=[
        pltpu.VMEM((CB + 1, CHUNK), jnp.int32),
        pltpu.VMEM((CB + 1, CHUNK), jnp.int32),
        pltpu.VMEM((4, CHUNK, DH), jnp.float32),
        pltpu.VMEM_SHARED((N_ACC, DH), jnp.float32),
        pltpu.VMEM_SHARED((N_ACC, DH), jnp.float32),
        pltpu.SemaphoreType.DMA,
        pltpu.SemaphoreType.DMA,
        pltpu.SemaphoreType.DMA,
        pltpu.SemaphoreType.DMA,
        pltpu.SemaphoreType.DMA,
        pltpu.SemaphoreType.DMA,
        pltpu.SemaphoreType.DMA,
        pltpu.SemaphoreType.DMA,
    ],
    compiler_params=_sc_params,
)
def _sc_aggregate(g_hbm, e_hbm, zeros_hbm, out_hbm,
                  src_v, dst_v, rows4, g_sp, acc,
                  gs0, gs1, gs2, gs3, ss0, ss1, ss2, ss3):
    cid = lax.axis_index("c")
    sid = lax.axis_index("s")
    wid = sid * NC + cid
    r0 = sid * RPT
    pltpu.sync_copy(zeros_hbm.at[pl.ds(r0, RPT)], acc.at[pl.ds(r0, RPT)])
    pltpu.sync_copy(g_hbm.at[pl.ds(r0, RPT)], g_sp.at[pl.ds(r0, RPT)])
    pltpu.sync_copy(e_hbm.at[pl.ds(wid * CB, CB)], src_v.at[pl.ds(0, CB)])
    pltpu.sync_copy(e_hbm.at[pl.ds(DROW + wid * CB, CB)],
                    dst_v.at[pl.ds(0, CB)])

    @pl.when(wid < 4)
    def _():
        pltpu.sync_copy(e_hbm.at[pl.ds(NW * CB + wid, 1)],
                        src_v.at[pl.ds(CB, 1)])
        pltpu.sync_copy(e_hbm.at[pl.ds(DROW + NW * CB + wid, 1)],
                        dst_v.at[pl.ds(CB, 1)])

    plsc.subcore_barrier()

    gsems = (gs0, gs1, gs2, gs3)
    ssems = (ss0, ss1, ss2, ss3)
    for b in range(4):
        pltpu.async_copy(g_sp.at[src_v.at[b]], rows4.at[b], gsems[b])

    def body(i, carry):
        j0 = 4 * i
        for b in range(4):
            j = j0 + b
            pltpu.make_async_copy(
                g_sp.at[src_v.at[j]], rows4.at[b], gsems[b]).wait()
            pltpu.async_copy(rows4.at[b], acc.at[dst_v.at[j]], ssems[b],
                             add=True)
        for b in range(4):
            j = j0 + b
            pltpu.make_async_copy(
                rows4.at[b], acc.at[dst_v.at[j]], ssems[b]).wait()

            @pl.when(j + 4 < CB)
            def _():
                pltpu.async_copy(g_sp.at[src_v.at[j + 4]], rows4.at[b],
                                 gsems[b])
        return carry

    lax.fori_loop(0, CB // 4, body, 0)
    for b in range(2):
        j = (CB // 4) * 4 + b
        pltpu.make_async_copy(
            g_sp.at[src_v.at[j]], rows4.at[b], gsems[b]).wait()
        pltpu.async_copy(rows4.at[b], acc.at[dst_v.at[j]], ssems[b],
                         add=True)
    for b in range(2):
        j = (CB // 4) * 4 + b
        pltpu.make_async_copy(
            rows4.at[b], acc.at[dst_v.at[j]], ssems[b]).wait()

    @pl.when(wid < 4)
    def _():
        pltpu.sync_copy(g_sp.at[src_v.at[CB]], rows4.at[0])
        pltpu.sync_copy(rows4.at[0], acc.at[dst_v.at[CB]], add=True)

    plsc.subcore_barrier()
    pltpu.sync_copy(acc.at[pl.ds(r0, RPT)], out_hbm.at[cid, pl.ds(r0, RPT)])


def _scale_body(xr_ref, w1b_ref, degp_ref, dinv_ref, g_ref):
    h_p = jnp.dot(xr_ref[...], w1b_ref[...],
                  preferred_element_type=jnp.float32)
    deg = degp_ref[:NAP, :] + degp_ref[NAP:, :] + 1.0
    dinv = lax.rsqrt(deg)
    dinv_ref[...] = dinv
    g_ref[:NP, :] = h_p * dinv[:NP, :]


def _mid_body(aggp_ref, g1_ref, dinv_ref, b1_ref, w2b_ref, g2_ref):
    s = aggp_ref[:NAP, :] + aggp_ref[NAP:, :] + g1_ref[...]
    a1 = dinv_ref[...] * s + b1_ref[...]
    h = jnp.maximum(a1, 0.0)
    h2 = jnp.dot(h, w2b_ref[...], preferred_element_type=jnp.float32)
    g2_ref[:NP, :] = (h2 * dinv_ref[...])[:NP, :]


def _out_body(aggp_ref, g2_ref, dinv_ref, b2_ref, sb_ref, o_ref):
    s = (aggp_ref[:NP, :] + aggp_ref[NAP:NAP + NP, :] + g2_ref[:NP, :])
    a = dinv_ref[:NP, :] * s + b2_ref[...]
    lanes = lax.broadcasted_iota(jnp.int32, (NP, 128), 1)
    m = a
    for k in (1, 2, 4, 8):
        up = jnp.roll(m, -k, axis=1)
        dn = jnp.roll(m, k, axis=1)
        m = jnp.maximum(m, jnp.where((lanes & k) == 0, up, dn))
    z = a - m
    ez = jnp.exp(z)
    gs = jnp.dot(ez, sb_ref[...], preferred_element_type=jnp.float32)
    o_ref[...] = z - jnp.log(gs)


_f32 = jnp.float32


def kernel(x, edge_index, W1, b1, W2, b2):
    e2 = edge_index.astype(jnp.int32).reshape(ER, CHUNK)
    zeros_acc = jnp.zeros((N_ACC, DH), _f32)
    ones_blk = jnp.ones((CHUNK, DH), _f32)
    W1B = jnp.kron(jnp.eye(PK, dtype=_f32), W1)
    W2B = jnp.kron(jnp.eye(PK, dtype=_f32), W2)
    SB = jnp.kron(jnp.eye(PK, dtype=_f32), jnp.ones((DH, DH), _f32))
    b1B = jnp.tile(b1, (PK,)).reshape(1, 128)
    b2B = jnp.tile(b2, (PK,)).reshape(1, 128)
    x_r = x.reshape(NP, PK * D_IN)

    degp = _sc_degree(e2, ones_blk, zeros_acc)
    degp_p = degp.reshape(NC * NAP, 128)

    dinv_p, g1_p = pl.pallas_call(
        _scale_body,
        out_shape=(jax.ShapeDtypeStruct((NAP, 128), _f32),
                   jax.ShapeDtypeStruct((NAP, 128), _f32)),
    )(x_r, W1B, degp_p)

    aggp1 = _sc_aggregate(g1_p.reshape(N_ACC, DH), e2, zeros_acc)

    g2_p = pl.pallas_call(
        _mid_body,
        out_shape=jax.ShapeDtypeStruct((NAP, 128), _f32),
    )(aggp1.reshape(NC * NAP, 128), g1_p, dinv_p, b1B, W2B)

    aggp2 = _sc_aggregate(g2_p.reshape(N_ACC, DH), e2, zeros_acc)

    out_p = pl.pallas_call(
        _out_body,
        out_shape=jax.ShapeDtypeStruct((NP, 128), _f32),
    )(aggp2.reshape(NC * NAP, 128), g2_p, dinv_p, b2B, SB)

    return out_p.reshape(N, DH)


# speedup vs baseline: 1.0330x; 1.0330x over previous
import functools

import jax
import jax.numpy as jnp
from jax import lax
from jax.experimental import pallas as pl
from jax.experimental.pallas import tpu as pltpu
from jax.experimental.pallas import tpu_sc as plsc

N = 10000
E = 320000
D_IN = 128
DH = 16
NC = 2
NS = 16
NW = NC * NS
CHUNK = 128
ER = 2 * E // CHUNK
CB = 78
DROW = E // CHUNK
N_ACC = 10112
RPT = N_ACC // NS
PK = 8
NP = N // PK
NAP = N_ACC // PK

_mesh = plsc.VectorSubcoreMesh(core_axis_name="c", subcore_axis_name="s")
_acc_ty = jax.ShapeDtypeStruct((NC, N_ACC, DH), jnp.float32)
_sc_params = pltpu.CompilerParams(use_tc_tiling_on_sc=False)


@functools.partial(
    pl.kernel,
    out_type=_acc_ty,
    mesh=_mesh,
    scratch_types=[
        pltpu.VMEM((CB + 1, CHUNK), jnp.int32),
        pltpu.VMEM((CHUNK, DH), jnp.float32),
        pltpu.VMEM_SHARED((N_ACC, DH), jnp.float32),
        pltpu.SemaphoreType.DMA,
        pltpu.SemaphoreType.DMA,
    ],
    compiler_params=_sc_params,
)
def _sc_degree(e_hbm, ones_hbm, zeros_hbm, out_hbm, dst_v, ones_v, acc,
               ss0, ss1):
    cid = lax.axis_index("c")
    sid = lax.axis_index("s")
    wid = sid * NC + cid
    r0 = sid * RPT
    pltpu.sync_copy(zeros_hbm.at[pl.ds(r0, RPT)], acc.at[pl.ds(r0, RPT)])
    pltpu.sync_copy(e_hbm.at[pl.ds(DROW + wid * CB, CB)],
                    dst_v.at[pl.ds(0, CB)])

    @pl.when(wid < 4)
    def _():
        pltpu.sync_copy(e_hbm.at[pl.ds(DROW + NW * CB + wid, 1)],
                        dst_v.at[pl.ds(CB, 1)])

    pltpu.sync_copy(ones_hbm, ones_v)
    plsc.subcore_barrier()

    sems = (ss0, ss1)
    for b in range(2):
        pltpu.async_copy(ones_v, acc.at[dst_v.at[b]], sems[b], add=True)

    def body(i, carry):
        j0 = 2 * i
        for b in range(2):
            j = j0 + b
            pltpu.make_async_copy(
                ones_v, acc.at[dst_v.at[j - 2]], sems[b]).wait()
            pltpu.async_copy(ones_v, acc.at[dst_v.at[j]], sems[b], add=True)
        return carry

    lax.fori_loop(1, CB // 2, body, 0)
    for b in range(2):
        pltpu.make_async_copy(
            ones_v, acc.at[dst_v.at[CB - 2 + b]], sems[b]).wait()

    @pl.when(wid < 4)
    def _():
        pltpu.sync_copy(ones_v, acc.at[dst_v.at[CB]], add=True)

    plsc.subcore_barrier()
    pltpu.sync_copy(acc.at[pl.ds(r0, RPT)], out_hbm.at[cid, pl.ds(r0, RPT)])


@functools.partial(
    pl.kernel,
    out_type=_acc_ty,
    mesh=_mesh,
    scratch_types=[
        pltpu.VMEM((CB + 1, CHUNK), jnp.int32),
        pltpu.VMEM((CB + 1, CHUNK), jnp.int32),
        pltpu.VMEM((2, CHUNK, DH), jnp.float32),
        pltpu.VMEM_SHARED((N_ACC, DH), jnp.float32),
        pltpu.VMEM_SHARED((N_ACC, DH), jnp.float32),
        pltpu.SemaphoreType.DMA,
        pltpu.SemaphoreType.DMA,
    ],
    compiler_params=_sc_params,
)
def _sc_aggregate(g_hbm, e_hbm, zeros_hbm, out_hbm,
                  src_v, dst_v, rows2, g_sp, acc, ss0, ss1):
    cid = lax.axis_index("c")
    sid = lax.axis_index("s")
    wid = sid * NC + cid
    r0 = sid * RPT
    pltpu.sync_copy(zeros_hbm.at[pl.ds(r0, RPT)], acc.at[pl.ds(r0, RPT)])
    pltpu.sync_copy(g_hbm.at[pl.ds(r0, RPT)], g_sp.at[pl.ds(r0, RPT)])
    pltpu.sync_copy(e_hbm.at[pl.ds(wid * CB, CB)], src_v.at[pl.ds(0, CB)])
    pltpu.sync_copy(e_hbm.at[pl.ds(DROW + wid * CB, CB)],
                    dst_v.at[pl.ds(0, CB)])

    @pl.when(wid < 4)
    def _():
        pltpu.sync_copy(e_hbm.at[pl.ds(NW * CB + wid, 1)],
                        src_v.at[pl.ds(CB, 1)])
        pltpu.sync_copy(e_hbm.at[pl.ds(DROW + NW * CB + wid, 1)],
                        dst_v.at[pl.ds(CB, 1)])

    plsc.subcore_barrier()

    sems = (ss0, ss1)
    for b in range(2):
        pltpu.sync_copy(g_sp.at[src_v.at[b]], rows2.at[b])
        pltpu.async_copy(rows2.at[b], acc.at[dst_v.at[b]], sems[b], add=True)

    def body(i, carry):
        j0 = 2 * i
        for b in range(2):
            j = j0 + b
            pltpu.make_async_copy(
                rows2.at[b], acc.at[dst_v.at[j - 2]], sems[b]).wait()
            pltpu.sync_copy(g_sp.at[src_v.at[j]], rows2.at[b])
            pltpu.async_copy(rows2.at[b], acc.at[dst_v.at[j]], sems[b],
                             add=True)
        return carry

    lax.fori_loop(1, CB // 2, body, 0)
    for b in range(2):
        pltpu.make_async_copy(
            rows2.at[b], acc.at[dst_v.at[CB - 2 + b]], sems[b]).wait()

    @pl.when(wid < 4)
    def _():
        pltpu.sync_copy(g_sp.at[src_v.at[CB]], rows2.at[0])
        pltpu.sync_copy(rows2.at[0], acc.at[dst_v.at[CB]], add=True)

    plsc.subcore_barrier()
    pltpu.sync_copy(acc.at[pl.ds(r0, RPT)], out_hbm.at[cid, pl.ds(r0, RPT)])


def _scale_body(xr_ref, w1b_ref, degp_ref, dinv_ref, g_ref):
    h_p = jnp.dot(xr_ref[...], w1b_ref[...],
                  preferred_element_type=jnp.float32)
    deg = degp_ref[:NAP, :] + degp_ref[NAP:, :] + 1.0
    dinv = lax.rsqrt(deg)
    dinv_ref[...] = dinv
    g_ref[:NP, :] = h_p * dinv[:NP, :]


def _mid_body(aggp_ref, g1_ref, dinv_ref, b1_ref, w2b_ref, g2_ref):
    s = aggp_ref[:NAP, :] + aggp_ref[NAP:, :] + g1_ref[...]
    a1 = dinv_ref[...] * s + b1_ref[...]
    h = jnp.maximum(a1, 0.0)
    h2 = jnp.dot(h, w2b_ref[...], preferred_element_type=jnp.float32)
    g2_ref[:NP, :] = (h2 * dinv_ref[...])[:NP, :]


def _out_body(aggp_ref, g2_ref, dinv_ref, b2_ref, sb_ref, o_ref):
    s = (aggp_ref[:NP, :] + aggp_ref[NAP:NAP + NP, :] + g2_ref[:NP, :])
    a = dinv_ref[:NP, :] * s + b2_ref[...]
    lanes = lax.broadcasted_iota(jnp.int32, (NP, 128), 1)
    m = a
    for k in (1, 2, 4, 8):
        up = jnp.roll(m, -k, axis=1)
        dn = jnp.roll(m, k, axis=1)
        m = jnp.maximum(m, jnp.where((lanes & k) == 0, up, dn))
    z = a - m
    ez = jnp.exp(z)
    gs = jnp.dot(ez, sb_ref[...], preferred_element_type=jnp.float32)
    o_ref[...] = z - jnp.log(gs)


_f32 = jnp.float32


def kernel(x, edge_index, W1, b1, W2, b2):
    e2 = edge_index.astype(jnp.int32).reshape(ER, CHUNK)
    zeros_acc = jnp.zeros((N_ACC, DH), _f32)
    ones_blk = jnp.ones((CHUNK, DH), _f32)
    W1B = jnp.kron(jnp.eye(PK, dtype=_f32), W1)
    W2B = jnp.kron(jnp.eye(PK, dtype=_f32), W2)
    SB = jnp.kron(jnp.eye(PK, dtype=_f32), jnp.ones((DH, DH), _f32))
    b1B = jnp.tile(b1, (PK,)).reshape(1, 128)
    b2B = jnp.tile(b2, (PK,)).reshape(1, 128)
    x_r = x.reshape(NP, PK * D_IN)

    degp = _sc_degree(e2, ones_blk, zeros_acc)
    degp_p = degp.reshape(NC * NAP, 128)

    dinv_p, g1_p = pl.pallas_call(
        _scale_body,
        out_shape=(jax.ShapeDtypeStruct((NAP, 128), _f32),
                   jax.ShapeDtypeStruct((NAP, 128), _f32)),
    )(x_r, W1B, degp_p)

    aggp1 = _sc_aggregate(g1_p.reshape(N_ACC, DH), e2, zeros_acc)

    g2_p = pl.pallas_call(
        _mid_body,
        out_shape=jax.ShapeDtypeStruct((NAP, 128), _f32),
    )(aggp1.reshape(NC * NAP, 128), g1_p, dinv_p, b1B, W2B)

    aggp2 = _sc_aggregate(g2_p.reshape(N_ACC, DH), e2, zeros_acc)

    out_p = pl.pallas_call(
        _out_body,
        out_shape=jax.ShapeDtypeStruct((NP, 128), _f32),
    )(aggp2.reshape(NC * NAP, 128), g2_p, dinv_p, b2B, SB)

    return out_p.reshape(N, DH)
